# baseline (speedup 1.0000x reference)
import functools

import jax
import jax.numpy as jnp
import numpy as np
from jax import lax
from jax.experimental import pallas as pl
from jax.experimental.pallas import tpu as pltpu
from jax.experimental.pallas import tpu_sc as plsc

_ORDERS = (2, 3)
_NUM_HEADS = 8
_EMBED = 1280
_HIDDEN = 2048
_KSIZE = 4
_DIL = 3
_SLOT = 80
_NT = 16
_P = 31253
_M16 = 3030
_ADJ = 7482
_BS = 512


def _floormod_p(h):
    hi = lax.shift_right_logical(h, 16)
    lo = jnp.bitwise_and(h, 0xFFFF)
    hi = jnp.where(hi >= _P, hi - _P, hi)
    hi = jnp.where(hi >= _P, hi - _P, hi)
    t1 = hi * _M16 + lo
    v = lax.shift_right_logical(t1, 16) * _M16 + jnp.bitwise_and(t1, 0xFFFF)
    q = (v.astype(jnp.float32) * jnp.float32(1.0 / _P)).astype(jnp.int32)
    r = v - q * _P
    r = jnp.where(r < 0, r + _P, r)
    r = jnp.where(r >= _P, r - _P, r)
    r = r + jnp.where(h < 0, _ADJ, 0)
    return jnp.where(r >= _P, r - _P, r)


def _sc_body(n_tok, n_workers, chunk, row_len,
             ids_hbm, map_hbm, seeds_hbm, tab_hbm, out_hbm,
             seeds_v, ids_v, can_v, idx_v, row_v, sem, sem2):
    cid = lax.axis_index("c")
    sid = lax.axis_index("s")
    nc = n_workers // 16
    wid = sid * nc + cid
    base = wid * chunk

    pltpu.sync_copy(seeds_hbm, seeds_v)

    @pl.when(base == 0)
    def _():
        ids_v[pl.ds(0, 16)] = jnp.zeros((16,), jnp.int32)
        pltpu.sync_copy(ids_hbm.at[pl.ds(0, chunk)], ids_v.at[pl.ds(16, chunk)])

    @pl.when(base > 0)
    def _():
        pltpu.sync_copy(ids_hbm.at[pl.ds(base - 16, chunk + 16)], ids_v)

    n_full = (chunk + 16) // 128
    for j in range(n_full):
        pltpu.async_copy(map_hbm.at[ids_v.at[pl.ds(j * 128, 128)]],
                         can_v.at[pl.ds(j * 128, 128)], sem).wait()
    rem = (chunk + 16) - n_full * 128
    if rem:
        pltpu.async_copy(map_hbm.at[ids_v.at[pl.ds(n_full * 128, rem)]],
                         can_v.at[pl.ds(n_full * 128, rem)], sem).wait()

    @pl.when(base % row_len == 0)
    def _():
        lane = lax.iota(jnp.int32, 16)
        c16 = can_v[pl.ds(0, 16)]
        can_v[pl.ds(0, 16)] = jnp.where(lane >= 14, 0, c16)

    seeds = [[seeds_v[j * _NUM_HEADS + k, :] for k in range(_NUM_HEADS)]
             for j in range(3)]

    def hash_step(p, carry):
        o = p * 16
        c0 = can_v[pl.ds(o + 16, 16)]
        c1 = can_v[pl.ds(o + 15, 16)]
        c2 = can_v[pl.ds(o + 14, 16)]
        t = 0
        for n in _ORDERS:
            for k in range(_NUM_HEADS):
                if n == 2:
                    h = (c1 * seeds[0][k]) ^ (c0 * seeds[1][k])
                else:
                    h = (c2 * seeds[0][k]) ^ (c1 * seeds[1][k]) ^ (c0 * seeds[2][k])
                idx_v[t, pl.ds(o, 16)] = _floormod_p(h) + t * _P
                t += 1
        return carry

    lax.fori_loop(0, chunk // 16, hash_step, 0)

    for t in range(_NT):
        for hh in range(chunk // 128):
            idxs = idx_v.at[t, pl.ds(hh * 128, 128)]
            pltpu.async_copy(tab_hbm.at[idxs], row_v, sem2).wait()
            pltpu.sync_copy(row_v,
                            out_hbm.at[pl.ds(base + hh * 128, 128),
                                       pl.ds(t * _SLOT, _SLOT)])


def _sc_gather(ids_flat, mapping, seeds, tab_flat, row_len):
    n_tok = ids_flat.shape[0]
    info = plsc.get_sparse_core_info()
    n_workers = info.num_cores * info.num_subcores
    chunk = n_tok // n_workers
    mesh = plsc.VectorSubcoreMesh(core_axis_name="c", subcore_axis_name="s")
    f = pl.kernel(
        functools.partial(_sc_body, n_tok, n_workers, chunk, row_len),
        out_type=jax.ShapeDtypeStruct((n_tok, _EMBED), jnp.float32),
        mesh=mesh,
        compiler_params=pltpu.CompilerParams(use_tc_tiling_on_sc=False),
        scratch_types=[
            pltpu.VMEM((3 * _NUM_HEADS, 16), jnp.int32),
            pltpu.VMEM((chunk + 16,), jnp.int32),
            pltpu.VMEM((chunk + 16,), jnp.int32),
            pltpu.VMEM((_NT, chunk), jnp.int32),
            pltpu.VMEM((128, _SLOT), jnp.float32),
            pltpu.SemaphoreType.DMA,
            pltpu.SemaphoreType.DMA,
        ],
    )
    return f(ids_flat, mapping, seeds, tab_flat)


def _dense_body(blocks_per_row, h_ref, m_ref, wk_ref, wv_ref, cw_ref,
                gg_ref, gc_ref, o_ref, state_ref):
    i = pl.program_id(0)

    @pl.when(i % blocks_per_row == 0)
    def _():
        state_ref[...] = jnp.zeros_like(state_ref)

    h = h_ref[...]
    hn = h * lax.rsqrt(jnp.mean(h * h, axis=-1, keepdims=True) + 1e-6)
    hn = hn * gg_ref[...]
    keys = jnp.dot(hn.astype(jnp.bfloat16), wk_ref[...],
                   preferred_element_type=jnp.float32)
    mem = m_ref[...]
    prod = keys * mem
    r1 = lax.broadcasted_iota(jnp.int32, (_EMBED, _NT), 0)
    c1 = lax.broadcasted_iota(jnp.int32, (_EMBED, _NT), 1) * _SLOT
    seg = ((r1 >= c1) & (r1 < c1 + _SLOT)).astype(jnp.float32)
    r2 = lax.broadcasted_iota(jnp.int32, (_NT, _EMBED), 1)
    c2 = lax.broadcasted_iota(jnp.int32, (_NT, _EMBED), 0) * _SLOT
    seg_t = ((r2 >= c2) & (r2 < c2 + _SLOT)).astype(jnp.float32)
    logits = jnp.dot(prod, seg, preferred_element_type=jnp.float32)
    gate = jax.nn.sigmoid(logits * (1.0 / np.sqrt(float(_SLOT))))
    gate_full = jnp.dot(gate, seg_t, preferred_element_type=jnp.float32)
    gated = gate_full * mem
    val = jnp.dot(gated.astype(jnp.bfloat16), wv_ref[...],
                  preferred_element_type=jnp.float32)
    xn = val * lax.rsqrt(jnp.mean(val * val, axis=-1, keepdims=True) + 1e-6)
    xn = xn * gc_ref[...]
    pad = (_KSIZE - 1) * _DIL
    xcat = jnp.concatenate([state_ref[...], xn], axis=0)
    cw = cw_ref[...]
    conv = xcat[0:_BS] * cw[0:1]
    for k in range(1, _KSIZE):
        conv = conv + xcat[k * _DIL:k * _DIL + _BS] * cw[k:k + 1]
    state_ref[...] = xn[_BS - pad:_BS]
    o_ref[...] = conv * jax.nn.sigmoid(conv) + val + h


def _dense(h2, mem, wk, wv, cwt, gg, gc, blocks_per_row):
    n_tok = h2.shape[0]
    grid = (n_tok // _BS,)
    return pl.pallas_call(
        functools.partial(_dense_body, blocks_per_row),
        grid=grid,
        in_specs=[
            pl.BlockSpec((_BS, _HIDDEN), lambda i: (i, 0)),
            pl.BlockSpec((_BS, _EMBED), lambda i: (i, 0)),
            pl.BlockSpec((_HIDDEN, _EMBED), lambda i: (0, 0)),
            pl.BlockSpec((_EMBED, _HIDDEN), lambda i: (0, 0)),
            pl.BlockSpec((_KSIZE, _HIDDEN), lambda i: (0, 0)),
            pl.BlockSpec((1, _HIDDEN), lambda i: (0, 0)),
            pl.BlockSpec((1, _HIDDEN), lambda i: (0, 0)),
        ],
        out_specs=pl.BlockSpec((_BS, _HIDDEN), lambda i: (i, 0)),
        out_shape=jax.ShapeDtypeStruct((n_tok, _HIDDEN), jnp.float32),
        scratch_shapes=[pltpu.VMEM(((_KSIZE - 1) * _DIL, _HIDDEN), jnp.float32)],
    )(h2, mem, wk, wv, cwt, gg, gc)


def kernel(input_ids, hidden_states, mapping, hash_seeds, tables, W_k, W_v,
           conv_w, g_gate, g_conv):
    B, S = input_ids.shape
    n_tok = B * S
    ids = input_ids.reshape(n_tok).astype(jnp.int32)
    seeds_b = jnp.broadcast_to(
        hash_seeds.astype(jnp.int32).reshape(3 * _NUM_HEADS, 1),
        (3 * _NUM_HEADS, 16))
    mem = _sc_gather(ids, mapping.astype(jnp.int32), seeds_b,
                     tables.reshape(_NT * _P, _SLOT), S)
    h2 = hidden_states.reshape(n_tok, _HIDDEN)
    y = _dense(h2, mem,
               W_k.astype(jnp.bfloat16), W_v.astype(jnp.bfloat16),
               conv_w.T, g_gate.reshape(1, _HIDDEN),
               g_conv.reshape(1, _HIDDEN), S // _BS)
    return y.reshape(B, S, _HIDDEN)
